# Initial kernel scaffold; baseline (speedup 1.0000x reference)
import jax
import jax.numpy as jnp
from jax.experimental import pallas as pl


def kernel(x, position, value_table):
    raise NotImplementedError("write your pallas kernel here")


# speedup vs baseline: 4.4228x; 4.4228x over previous
import jax
import jax.numpy as jnp
from jax.experimental import pallas as pl


def _fire_kernel(x_ref, pos_ref, tab_ref, out_ref):
    b = pl.program_id(0)
    L = tab_ref.shape[0]
    P = x_ref.shape[1]
    row = x_ref[pl.ds(b, 1), :]
    idx = (row * float(L - 1)).astype(jnp.int32)
    levels = jax.lax.broadcasted_iota(jnp.int32, (L, P), 0)
    onehot = (levels == idx).astype(jnp.bfloat16)
    q = jnp.dot(onehot, pos_ref[...], preferred_element_type=jnp.float32)
    multiset = jnp.sum(q * tab_ref[...], axis=0)
    out_ref[pl.ds(b, 1), :] = jnp.where(multiset > 0, 1.0, -1.0).astype(
        jnp.float32
    )[None, :]


def kernel(x, position, value_table):
    B = x.shape[0]
    flat = x.reshape(B, -1)
    P = flat.shape[1]
    L, D = value_table.shape
    pos_bf = position.astype(jnp.bfloat16)
    return pl.pallas_call(
        _fire_kernel,
        grid=(B,),
        in_specs=[
            pl.BlockSpec((B, P), lambda b: (0, 0)),
            pl.BlockSpec((P, D), lambda b: (0, 0)),
            pl.BlockSpec((L, D), lambda b: (0, 0)),
        ],
        out_specs=pl.BlockSpec((B, D), lambda b: (0, 0)),
        out_shape=jax.ShapeDtypeStruct((B, D), jnp.float32),
    )(flat, pos_bf, value_table)


# speedup vs baseline: 7.4833x; 1.6920x over previous
import jax
import jax.numpy as jnp
from jax.experimental import pallas as pl


def _fire_kernel(x_ref, pos_ref, tab_ref, out_ref):
    b = pl.program_id(0)
    L = tab_ref.shape[0]
    P = x_ref.shape[1]
    row = x_ref[pl.ds(b, 1), :]
    idx = (row * float(L - 1)).astype(jnp.int32)
    levels = jax.lax.broadcasted_iota(jnp.int32, (L, P), 0)
    onehot = (levels == idx).astype(jnp.float8_e4m3fn)
    q = jnp.dot(onehot, pos_ref[...], preferred_element_type=jnp.float32)
    multiset = jnp.sum(q * tab_ref[...], axis=0)
    out_ref[pl.ds(b, 1), :] = jnp.where(multiset > 0, 1.0, -1.0).astype(
        jnp.float32
    )[None, :]


def kernel(x, position, value_table):
    B = x.shape[0]
    flat = x.reshape(B, -1)
    P = flat.shape[1]
    L, D = value_table.shape
    pos_bf = position.astype(jnp.float8_e4m3fn)
    return pl.pallas_call(
        _fire_kernel,
        grid=(B,),
        in_specs=[
            pl.BlockSpec((B, P), lambda b: (0, 0)),
            pl.BlockSpec((P, D), lambda b: (0, 0)),
            pl.BlockSpec((L, D), lambda b: (0, 0)),
        ],
        out_specs=pl.BlockSpec((B, D), lambda b: (0, 0)),
        out_shape=jax.ShapeDtypeStruct((B, D), jnp.float32),
    )(flat, pos_bf, value_table)
